# baseline (speedup 1.0000x reference)
import functools

import jax
import jax.numpy as jnp
from jax import lax
from jax.experimental import pallas as pl
from jax.experimental.pallas import tpu as pltpu
from jax.experimental.pallas import tpu_sc as plsc

N_NODES = 10000
N_EDGES = 320000
D_FEAT = 128

NC = 2
NS = 16
NW = NC * NS
LANES = 16

CHUNK = 128
K_CHUNKS = -(-N_EDGES // (NW * CHUNK))
K_CHUNKS += K_CHUNKS % 2
E_PAD = NW * K_CHUNKS * CHUNK
ACC_ROWS = 10240
STRIPE = ACC_ROWS // NS
STRIPE_BLKS = STRIPE // CHUNK


def _sc_segment_sum(x, packed3):
    mesh = plsc.VectorSubcoreMesh(core_axis_name="c", subcore_axis_name="s")

    @functools.partial(
        pl.kernel,
        mesh=mesh,
        out_type=jax.ShapeDtypeStruct((NC, ACC_ROWS, D_FEAT), jnp.float32),
        scratch_types=[
            pltpu.VMEM((K_CHUNKS, CHUNK), jnp.int32),
            pltpu.VMEM((2, CHUNK), jnp.int32),
            pltpu.VMEM((2, CHUNK), jnp.int32),
            pltpu.VMEM((CHUNK, D_FEAT), jnp.float32),
            pltpu.VMEM((CHUNK, D_FEAT), jnp.float32),
            pltpu.VMEM_SHARED((ACC_ROWS, D_FEAT), jnp.float32),
            pltpu.SemaphoreType.DMA,
            pltpu.SemaphoreType.DMA,
        ],
    )
    def k(x_hbm, pck_hbm, out_hbm, pidx, sidx, didx, rows0, rows1, acc,
          g0, g1):
        c = lax.axis_index("c")
        s = lax.axis_index("s")
        wid = s * NC + c

        h_idx = pltpu.async_copy(pck_hbm.at[wid], pidx, g0)

        @pl.loop(0, CHUNK)
        def _(r):
            @pl.loop(0, D_FEAT, step=LANES)
            def _(col):
                rows0.at[pl.ds(r, 1), pl.ds(col, LANES)][...] = jnp.zeros(
                    (1, LANES), jnp.float32)

        @pl.loop(0, STRIPE_BLKS)
        def _(b):
            pltpu.sync_copy(rows0, acc.at[pl.ds(s * STRIPE + b * CHUNK, CHUNK)])

        h_idx.wait()
        plsc.subcore_barrier()

        def unpack(j, slot):
            for col in range(0, CHUNK, LANES):
                v = pidx.at[pl.ds(j, 1), pl.ds(col, LANES)][...]
                sidx.at[pl.ds(slot, 1), pl.ds(col, LANES)][...] = (
                    lax.bitwise_and(v, jnp.int32(0xFFFF)))
                didx.at[pl.ds(slot, 1), pl.ds(col, LANES)][...] = (
                    lax.shift_right_logical(v, jnp.int32(16)))

        unpack(0, 0)
        pltpu.async_copy(x_hbm.at[sidx.at[0]], rows0, g0)

        @pl.loop(0, K_CHUNKS - 2, step=2)
        def _(j):
            unpack(j + 1, 1)
            pltpu.async_copy(x_hbm.at[sidx.at[1]], rows1, g1)
            pltpu.make_async_copy(x_hbm.at[sidx.at[0]], rows0, g0).wait()
            pltpu.sync_copy(rows0, acc.at[didx.at[0]], add=True)
            unpack(j + 2, 0)
            pltpu.async_copy(x_hbm.at[sidx.at[0]], rows0, g0)
            pltpu.make_async_copy(x_hbm.at[sidx.at[1]], rows1, g1).wait()
            pltpu.sync_copy(rows1, acc.at[didx.at[1]], add=True)

        unpack(K_CHUNKS - 1, 1)
        pltpu.async_copy(x_hbm.at[sidx.at[1]], rows1, g1)
        pltpu.make_async_copy(x_hbm.at[sidx.at[0]], rows0, g0).wait()
        pltpu.sync_copy(rows0, acc.at[didx.at[0]], add=True)
        pltpu.make_async_copy(x_hbm.at[sidx.at[1]], rows1, g1).wait()
        pltpu.sync_copy(rows1, acc.at[didx.at[1]], add=True)

        plsc.subcore_barrier()

        @pl.loop(0, STRIPE_BLKS)
        def _(b):
            base = s * STRIPE + b * CHUNK
            pltpu.sync_copy(acc.at[pl.ds(base, CHUNK)],
                            out_hbm.at[c, pl.ds(base, CHUNK)])

    return k(x, packed3)


def _combine(parts, W):
    blk = 1000

    def body(p_ref, w_ref, o_ref):
        o_ref[...] = (p_ref[0] + p_ref[1]) * w_ref[...]

    return pl.pallas_call(
        body,
        grid=(N_NODES // blk,),
        in_specs=[
            pl.BlockSpec((NC, blk, D_FEAT), lambda i: (0, i, 0)),
            pl.BlockSpec((1, D_FEAT), lambda i: (0, 0)),
        ],
        out_specs=pl.BlockSpec((blk, D_FEAT), lambda i: (i, 0)),
        out_shape=jax.ShapeDtypeStruct((N_NODES, D_FEAT), jnp.float32),
    )(parts, W)


def kernel(x, edge_index, W):
    src = edge_index[0]
    dst = edge_index[1]
    pad = E_PAD - N_EDGES
    src_p = jnp.concatenate([src, jnp.zeros((pad,), jnp.int32)])
    dst_p = jnp.concatenate([dst, jnp.full((pad,), N_NODES, jnp.int32)])
    packed3 = ((dst_p << 16) | src_p).reshape(NW, K_CHUNKS, CHUNK)
    parts = _sc_segment_sum(x, packed3)
    return _combine(parts, W)
